# Initial kernel scaffold; baseline (speedup 1.0000x reference)
import jax
import jax.numpy as jnp
from jax.experimental import pallas as pl


def kernel(x, wslab, gslab):
    raise NotImplementedError("write your pallas kernel here")


# speedup vs baseline: 5.0934x; 5.0934x over previous
import math

import numpy as np
import jax
import jax.numpy as jnp
from jax import lax
from jax.experimental import pallas as pl
from jax.experimental.pallas import tpu as pltpu

_DIM = 32
_HEADS = 4
_N_ORF = 16
_HM = 64
_NTOK = 17
_PAIR = 2 * _NTOK
_CPP = 48
_MLP = 128
_DEPTH = 2
_OUT_PAD = 128
_N_CLASSES = 10
_LN_EPS = 1e-6

_G_SUB = 16
_N_CHAIN = 1
_G_STEP = _G_SUB * _N_CHAIN
_R_SUB = _PAIR * _G_SUB

_W_FUSEDW, _W_FUSEDB = 0, 32
_W_PROJW, _W_FC1W, _W_FC2W = 40, 72, 104
_W_PROJB, _W_FC1B, _W_FC2B = 232, 233, 234
_W_LN1G, _W_LN1B, _W_LN2G, _W_LN2B = 235, 236, 237, 238
_G_PATCHW, _G_POSALL, _G_HEADW = 0, 48, 88
_G_HMASK, _G_NORMG, _G_NORMB, _G_HEADB = 388, 394, 395, 396


def _erf_poly(v):
    a1, a2, a3, a4, a5 = (0.254829592, -0.284496736, 1.421413741,
                          -1.453152027, 1.061405429)
    p = 0.3275911
    sgn = jnp.where(v >= 0.0, 1.0, -1.0)
    av = v * sgn
    t = 1.0 / (1.0 + p * av)
    poly = ((((a5 * t + a4) * t + a3) * t + a2) * t + a1) * t
    return sgn * (1.0 - poly * jnp.exp(-av * av))


def _mm(a, b):
    return jnp.dot(a, b, preferred_element_type=jnp.float32)


def _ln(v, g, b):
    mu = jnp.mean(v, axis=-1, keepdims=True)
    ms = jnp.mean(v * v, axis=-1, keepdims=True)
    var = ms - mu * mu
    return (v - mu) * lax.rsqrt(var + _LN_EPS) * g + b


def _chain(patches, pos, imask, sel, masks, w_ref, g_ref):
    f32 = jnp.float32
    ratio = 1.0 / math.sqrt(math.sqrt(float(_N_ORF)))
    lratio = math.log(ratio)
    reps = ratio * 1e-6
    featsel, numsel, densel = masks

    xv = pos + _mm(patches, g_ref[_G_PATCHW:_G_PATCHW + _CPP, 0:_DIM])

    for d in range(_DEPTH):
        h1 = _ln(xv, w_ref[d, _W_LN1G:_W_LN1G + 1, 0:_DIM],
                 w_ref[d, _W_LN1B:_W_LN1B + 1, 0:_DIM])
        fused = (_mm(h1, w_ref[d, _W_FUSEDW:_W_FUSEDW + _DIM, :])
                 + w_ref[d, _W_FUSEDB:_W_FUSEDB + 1, :])
        v1 = fused[:, 0:_HM]
        qd = fused[:, 64:128]
        kd = fused[:, 128:192]
        dq = fused[:, 192:256]
        dk = fused[:, 256:320]

        gmax = jnp.max(jnp.max(kd, axis=0, keepdims=True), axis=1,
                       keepdims=True)
        kp = jnp.exp(kd - dk - gmax + lratio) + reps

        qmax = jnp.max(qd, axis=-1, keepdims=True)
        qp = jnp.exp(qd - dq - qmax + lratio) + reps

        numden = jnp.zeros_like(qd)
        for h in range(_HEADS):
            ah = lax.dot_general(qp, kp * featsel[h],
                                 (((1,), (1,)), ((), ())),
                                 preferred_element_type=f32)
            ahm = jnp.where(imask != 0.0, ah, 0.0)
            numden = numden + _mm(ahm, v1 * numsel[h] + densel[h])
        attn = numden[:, 0:_DIM] * pl.reciprocal(
            numden[:, _DIM:2 * _DIM], approx=True)

        xv = (xv + _mm(attn, w_ref[d, _W_PROJW:_W_PROJW + _DIM, 0:_DIM])
              + w_ref[d, _W_PROJB:_W_PROJB + 1, 0:_DIM])

        h2 = _ln(xv, w_ref[d, _W_LN2G:_W_LN2G + 1, 0:_DIM],
                 w_ref[d, _W_LN2B:_W_LN2B + 1, 0:_DIM])
        m1 = (_mm(h2, w_ref[d, _W_FC1W:_W_FC1W + _DIM, 0:_MLP])
              + w_ref[d, _W_FC1B:_W_FC1B + 1, 0:_MLP])
        m1 = 0.5 * m1 * (1.0 + _erf_poly(m1 * (1.0 / math.sqrt(2.0))))
        xv = (xv + _mm(m1, w_ref[d, _W_FC2W:_W_FC2W + _MLP, 0:_DIM])
              + w_ref[d, _W_FC2B:_W_FC2B + 1, 0:_DIM])

    cls = _mm(sel, xv)
    cls_n = _ln(cls, g_ref[_G_NORMG:_G_NORMG + 1, 0:_DIM],
                g_ref[_G_NORMB:_G_NORMB + 1, 0:_DIM])
    return (_mm(cls_n, g_ref[_G_HEADW:_G_HEADW + _DIM, :])
            + g_ref[_G_HEADB:_G_HEADB + 1, :])


def _fwd_body(patches_ref, pos_ref, imask_ref, sel_ref, w_ref, g_ref, o_ref):
    f32 = jnp.float32
    lane = lax.broadcasted_iota(jnp.int32, (1, _HM), 1)
    featsel = [(lane // _N_ORF == h).astype(f32) for h in range(_HEADS)]
    numsel = [jnp.where((lane < _DIM) & (lane // (_DIM // _HEADS) == h),
                        1.0, 0.0) for h in range(_HEADS)]
    densel = [jnp.where((lane >= _DIM) &
                        ((lane - _DIM) // (_DIM // _HEADS) == h), 1.0, 0.0)
              for h in range(_HEADS)]
    masks = (featsel, numsel, densel)

    pos = pos_ref[...]
    imask = imask_ref[...]
    sel = sel_ref[...]
    nrow = 2 * _G_SUB
    for c in range(_N_CHAIN):
        out_c = _chain(patches_ref[c * _R_SUB:(c + 1) * _R_SUB, :],
                       pos, imask, sel, masks, w_ref, g_ref)
        o_ref[c * nrow:(c + 1) * nrow, :] = out_c


def kernel(x, wslab, gslab):
    nb, two, c, hh, ww = x.shape
    p = 4
    steps = nb // _G_STEP

    gh, gw = hh // p, ww // p
    patches = x.reshape(nb * two, c, gh, p, gw, p).transpose(0, 2, 4, 1, 3, 5)
    patches = patches.reshape(nb, two, gh * gw, c * p * p)
    patches = jnp.pad(patches, ((0, 0), (0, 0), (1, 0), (0, 0)))
    patches = patches.reshape(nb * two * _NTOK, c * p * p)

    posall = gslab[_G_POSALL:_G_POSALL + _PAIR, 0:_DIM]
    pos = jnp.tile(posall, (_G_SUB, 1))

    iid = np.arange(_R_SUB) // _NTOK
    imask = jnp.asarray((iid[:, None] == iid[None, :]).astype(np.float32))
    sel = jnp.asarray(
        (np.arange(_R_SUB)[None, :] ==
         np.arange(2 * _G_SUB)[:, None] * _NTOK).astype(np.float32))

    out = pl.pallas_call(
        _fwd_body,
        out_shape=jax.ShapeDtypeStruct((2 * nb, _OUT_PAD), jnp.float32),
        grid=(steps,),
        in_specs=[
            pl.BlockSpec((_PAIR * _G_STEP, _CPP), lambda i: (i, 0)),
            pl.BlockSpec((_R_SUB, _DIM), lambda i: (0, 0)),
            pl.BlockSpec((_R_SUB, _R_SUB), lambda i: (0, 0)),
            pl.BlockSpec((2 * _G_SUB, _R_SUB), lambda i: (0, 0)),
            pl.BlockSpec(wslab.shape, lambda i: (0, 0, 0)),
            pl.BlockSpec(gslab.shape, lambda i: (0, 0)),
        ],
        out_specs=pl.BlockSpec((2 * _G_STEP, _OUT_PAD), lambda i: (i, 0)),
        compiler_params=pltpu.CompilerParams(
            dimension_semantics=("parallel",)),
    )(patches, pos, imask, sel, wslab, gslab)

    return out.reshape(nb, two, _OUT_PAD)[:, :, :_N_CLASSES]
